# baseline (speedup 1.0000x reference)
import jax
import jax.numpy as jnp
from jax import lax
from jax.experimental import pallas as pl
from jax.experimental.pallas import tpu as pltpu
from jax.experimental.pallas import tpu_sc as plsc

WD = 1e-4
B = 16384
D = 64
L = 16
NC = 2
NS = 16
NW = NC * NS
BPW = B // NW
G = 16
NG = BPW // G
VROWS = 500000


def _sc_body(emb_u2, emb_i2, users, pos, neg, raw_u2, raw_i2,
             scores_out, sq_out,
             idxu, idxp, idxn, blku, blkp, blkn,
             bu, bp, bn, bru, brp, brn, scb, sqb, sem):
    cid = lax.axis_index("c")
    sid = lax.axis_index("s")
    wid = sid * NC + cid
    base = wid * BPW

    pltpu.sync_copy(users.at[pl.ds(base, BPW)], idxu)
    pltpu.sync_copy(pos.at[pl.ds(base, BPW)], idxp)
    pltpu.sync_copy(neg.at[pl.ds(base, BPW)], idxn)

    def group(c, sq_acc):
        s16 = pl.ds(c * G, G)
        ivu = idxu[s16]
        ivp = idxp[s16]
        ivn = idxn[s16]
        blku[...] = lax.shift_right_logical(ivu, 1)
        blkp[...] = lax.shift_right_logical(ivp, 1)
        blkn[...] = lax.shift_right_logical(ivn, 1)
        cps = [
            pltpu.async_copy(emb_u2.at[blku], bu, sem),
            pltpu.async_copy(emb_i2.at[blkp], bp, sem),
            pltpu.async_copy(emb_i2.at[blkn], bn, sem),
            pltpu.async_copy(raw_u2.at[blku], bru, sem),
            pltpu.async_copy(raw_i2.at[blkp], brp, sem),
            pltpu.async_copy(raw_i2.at[blkn], brn, sem),
        ]
        for cp in cps:
            cp.wait()

        sqv = sq_acc
        for j in range(G):
            ou = jnp.bitwise_and(ivu[j], 1) * D
            op = jnp.bitwise_and(ivp[j], 1) * D
            on = jnp.bitwise_and(ivn[j], 1) * D
            acc = jnp.zeros((L,), jnp.float32)
            for k in range(D // L):
                uv = bu[j, pl.ds(ou + k * L, L)]
                pv = bp[j, pl.ds(op + k * L, L)]
                nv = bn[j, pl.ds(on + k * L, L)]
                acc = acc + uv * (nv - pv)
                av = bru[j, pl.ds(ou + k * L, L)]
                bv = brp[j, pl.ds(op + k * L, L)]
                cv = brn[j, pl.ds(on + k * L, L)]
                sqv = sqv + av * av + bv * bv + cv * cv
            scb[j, pl.ds(0, L)] = acc
        pltpu.sync_copy(scb, scores_out.at[pl.ds(base + c * G, G)])
        return sqv

    sq_acc = lax.fori_loop(0, NG, group, jnp.zeros((L,), jnp.float32))
    sqb[...] = sq_acc
    pltpu.sync_copy(sqb, sq_out.at[pl.ds(wid * L, L)])


def _tc_body(sc_ref, sq_ref, out_ref):
    x = jnp.sum(sc_ref[:, 0:L], axis=1, keepdims=True)
    sp = jnp.maximum(x, 0.0) + jnp.log1p(jnp.exp(-jnp.abs(x)))
    reg = jnp.sum(sq_ref[...])
    out_ref[0, 0] = jnp.sum(sp) / B + (0.5 * WD / B) * reg


def kernel(emb_users, emb_items, users, pos_items, neg_items,
           raw_emb_users, raw_emb_items):
    users = users.astype(jnp.int32)
    pos_items = pos_items.astype(jnp.int32)
    neg_items = neg_items.astype(jnp.int32)
    emb_u2 = emb_users.reshape(VROWS, 128)
    emb_i2 = emb_items.reshape(VROWS, 128)
    raw_u2 = raw_emb_users.reshape(VROWS, 128)
    raw_i2 = raw_emb_items.reshape(VROWS, 128)

    mesh = plsc.VectorSubcoreMesh(
        core_axis_name="c", subcore_axis_name="s",
        num_cores=NC, num_subcores=NS)
    sc = pl.kernel(
        _sc_body,
        out_type=[
            jax.ShapeDtypeStruct((B, 128), jnp.float32),
            jax.ShapeDtypeStruct((NW * L,), jnp.float32),
        ],
        mesh=mesh,
        scratch_types=[
            pltpu.VMEM((BPW,), jnp.int32),
            pltpu.VMEM((BPW,), jnp.int32),
            pltpu.VMEM((BPW,), jnp.int32),
            pltpu.VMEM((G,), jnp.int32),
            pltpu.VMEM((G,), jnp.int32),
            pltpu.VMEM((G,), jnp.int32),
            pltpu.VMEM((G, 128), jnp.float32),
            pltpu.VMEM((G, 128), jnp.float32),
            pltpu.VMEM((G, 128), jnp.float32),
            pltpu.VMEM((G, 128), jnp.float32),
            pltpu.VMEM((G, 128), jnp.float32),
            pltpu.VMEM((G, 128), jnp.float32),
            pltpu.VMEM((G, 128), jnp.float32),
            pltpu.VMEM((L,), jnp.float32),
            pltpu.SemaphoreType.DMA,
        ],
    )
    scores, sq = sc(emb_u2, emb_i2, users, pos_items, neg_items,
                    raw_u2, raw_i2)

    out = pl.pallas_call(
        _tc_body,
        out_shape=jax.ShapeDtypeStruct((1, 1), jnp.float32),
        out_specs=pl.BlockSpec(memory_space=pltpu.SMEM),
    )(scores, sq.reshape(4, 128))
    return out[0, 0]
